# baseline (device time: 31805 ns/iter reference)
import jax
import jax.numpy as jnp
from jax import lax
from jax.experimental import pallas as pl
from jax.experimental.pallas import tpu as pltpu

B = 8
H = 8
D = 128
BS = 16
NB = 512
P_LOC = 512
T_LOC = P_LOC * BS


def kernel(Q, K, V, bt, lens):
    lens2 = lens.reshape(B, 1)

    def body(q_ref, k_hbm, v_hbm, bt_ref, lens_ref, out_ref,
             kh_buf, vh_buf, o_send, o_recv, ml_send, ml_recv,
             copy_sems, sem_o_send, sem_o_recv, sem_ml_send, sem_ml_recv):
        my_x = lax.axis_index("x")
        my_y = lax.axis_index("y")
        my_z = lax.axis_index("z")
        partner = (my_x, 1 - my_y, my_z)

        def start_head_copy(h):
            slot = h % 2
            kc = pltpu.make_async_copy(
                k_hbm.at[:, :, h, :], kh_buf.at[slot], copy_sems.at[slot, 0])
            vc = pltpu.make_async_copy(
                v_hbm.at[:, :, h, :], vh_buf.at[slot], copy_sems.at[slot, 1])
            kc.start()
            vc.start()
            return kc, vc

        def wait_head_copy(h):
            slot = h % 2
            pltpu.make_async_copy(
                k_hbm.at[:, :, h, :], kh_buf.at[slot], copy_sems.at[slot, 0]
            ).wait()
            pltpu.make_async_copy(
                v_hbm.at[:, :, h, :], vh_buf.at[slot], copy_sems.at[slot, 1]
            ).wait()

        start_head_copy(0)

        j_iota = lax.broadcasted_iota(jnp.int32, (B, NB), 1)
        valid = j_iota < lens_ref[...]
        bt_m = jnp.where(valid, bt_ref[...], -1)
        hi = bt_m >> 7
        lo = bt_m & 127
        hi_iota = my_y * 4 + lax.broadcasted_iota(jnp.int32, (B, NB, 4), 2)
        lo_iota = lax.broadcasted_iota(jnp.int32, (B, NB, 128), 2)
        match_hi = (hi[:, :, None] == hi_iota).astype(jnp.float32)
        match_lo = (lo[:, :, None] == lo_iota).astype(jnp.float32)
        w4 = lax.dot_general(
            match_hi, match_lo, (((1,), (1,)), ((0,), (0,))),
            preferred_element_type=jnp.float32)
        wt = jnp.broadcast_to(w4[:, :, :, None], (B, 4, 128, BS)).reshape(B, T_LOC)

        qs = q_ref[...][:, 0]
        scale = D ** -0.5

        m_cols = []
        l_cols = []
        o_heads = []
        for h in range(H):
            if h + 1 < H:
                start_head_copy(h + 1)
            wait_head_copy(h)
            slot = h % 2
            kh = kh_buf[slot].reshape(T_LOC, D)
            vh = vh_buf[slot].reshape(T_LOC, D)
            q_h = qs[:, h, :]
            s = lax.dot_general(
                q_h, kh, (((1,), (1,)), ((), ())),
                preferred_element_type=jnp.float32) * scale
            m_h = s.max(axis=1, keepdims=True)
            e = jnp.exp(s - m_h) * wt
            l_h = e.sum(axis=1, keepdims=True)
            o_h = lax.dot_general(
                e, vh, (((1,), (0,)), ((), ())),
                preferred_element_type=jnp.float32)
            m_cols.append(m_h)
            l_cols.append(l_h)
            o_heads.append(o_h[:, None, :])

        m = jnp.concatenate(m_cols, axis=1)
        l = jnp.concatenate(l_cols, axis=1)
        o = jnp.concatenate(o_heads, axis=1)

        o_send[...] = o
        ml_send[0] = m
        ml_send[1] = l

        barrier = pltpu.get_barrier_semaphore()
        pl.semaphore_signal(barrier, inc=1, device_id=partner,
                            device_id_type=pl.DeviceIdType.MESH)
        pl.semaphore_wait(barrier, 1)

        rdma_o = pltpu.make_async_remote_copy(
            src_ref=o_send, dst_ref=o_recv,
            send_sem=sem_o_send, recv_sem=sem_o_recv,
            device_id=partner, device_id_type=pl.DeviceIdType.MESH)
        rdma_ml = pltpu.make_async_remote_copy(
            src_ref=ml_send, dst_ref=ml_recv,
            send_sem=sem_ml_send, recv_sem=sem_ml_recv,
            device_id=partner, device_id_type=pl.DeviceIdType.MESH)
        rdma_o.start()
        rdma_ml.start()
        rdma_o.wait()
        rdma_ml.wait()

        m_o = ml_recv[0]
        l_o = ml_recv[1]
        m_g = jnp.maximum(m, m_o)
        c_s = jnp.exp(m - m_g)
        c_o = jnp.exp(m_o - m_g)
        l_g = l * c_s + l_o * c_o
        out = (o * c_s[:, :, None] + o_recv[...] * c_o[:, :, None]) / l_g[:, :, None]
        out_ref[...] = out.reshape(B, 1, H, D)

    return pl.pallas_call(
        body,
        out_shape=jax.ShapeDtypeStruct((B, 1, H, D), jnp.float32),
        in_specs=[
            pl.BlockSpec(memory_space=pltpu.VMEM),
            pl.BlockSpec(memory_space=pltpu.MemorySpace.HBM),
            pl.BlockSpec(memory_space=pltpu.MemorySpace.HBM),
            pl.BlockSpec(memory_space=pltpu.VMEM),
            pl.BlockSpec(memory_space=pltpu.VMEM),
        ],
        out_specs=pl.BlockSpec(memory_space=pltpu.VMEM),
        scratch_shapes=[
            pltpu.VMEM((2, P_LOC, BS, D), jnp.float32),
            pltpu.VMEM((2, P_LOC, BS, D), jnp.float32),
            pltpu.VMEM((B, H, D), jnp.float32),
            pltpu.VMEM((B, H, D), jnp.float32),
            pltpu.VMEM((2, B, H), jnp.float32),
            pltpu.VMEM((2, B, H), jnp.float32),
            pltpu.SemaphoreType.DMA((2, 2)),
            pltpu.SemaphoreType.DMA,
            pltpu.SemaphoreType.DMA,
            pltpu.SemaphoreType.DMA,
            pltpu.SemaphoreType.DMA,
        ],
        compiler_params=pltpu.CompilerParams(collective_id=0),
    )(Q, K, V, bt, lens2)


# device time: 24461 ns/iter; 1.3002x vs baseline; 1.3002x over previous
import jax
import jax.numpy as jnp
from jax import lax
from jax.experimental import pallas as pl
from jax.experimental.pallas import tpu as pltpu

B = 8
H = 8
D = 128
BS = 16
NB = 512
P_LOC = 512
R = 8
P_REP = P_LOC // R
T_REP = P_REP * BS
PK = D + 2


def kernel(Q, K, V, bt, lens):
    lens2 = lens.reshape(B, 1)

    def body(q_ref, k_hbm, v_hbm, bt_ref, lens_ref, out_ref,
             kh_buf, vh_buf, part_send, part_recv, part_g, gather_buf,
             kv_sems, y_send_sem, y_recv_sem, g_send_sems, g_recv_sems,
             self_sem):
        my_x = lax.axis_index("x")
        my_y = lax.axis_index("y")
        my_z = lax.axis_index("z")
        my_r = my_x * 4 + my_z
        partner = (my_x, 1 - my_y, my_z)
        p0 = my_r * P_REP

        barrier = pltpu.get_barrier_semaphore()
        pl.semaphore_signal(barrier, inc=1, device_id=partner,
                            device_id_type=pl.DeviceIdType.MESH)
        for idx in range(R - 1):
            rr = lax.rem(my_r + 1 + idx, R)
            pl.semaphore_signal(barrier, inc=1,
                                device_id=(rr // 4, my_y, lax.rem(rr, 4)),
                                device_id_type=pl.DeviceIdType.MESH)

        def start_head_copy(h):
            slot = h % 2
            pltpu.make_async_copy(
                k_hbm.at[pl.ds(p0, P_REP), :, h, :], kh_buf.at[slot],
                kv_sems.at[slot, 0]).start()
            pltpu.make_async_copy(
                v_hbm.at[pl.ds(p0, P_REP), :, h, :], vh_buf.at[slot],
                kv_sems.at[slot, 1]).start()

        def wait_head_copy(h):
            slot = h % 2
            pltpu.make_async_copy(
                k_hbm.at[pl.ds(p0, P_REP), :, h, :], kh_buf.at[slot],
                kv_sems.at[slot, 0]).wait()
            pltpu.make_async_copy(
                v_hbm.at[pl.ds(p0, P_REP), :, h, :], vh_buf.at[slot],
                kv_sems.at[slot, 1]).wait()

        start_head_copy(0)

        j_iota = lax.broadcasted_iota(jnp.int32, (B, NB), 1)
        valid = j_iota < lens_ref[...]
        bt_m = jnp.where(valid, bt_ref[...], -1)
        sel = (bt_m >> 6 == my_y * 8 + my_r).astype(jnp.float32)
        lo_iota = lax.broadcasted_iota(jnp.int32, (B, NB, P_REP), 2)
        match_lo = ((bt_m & 63)[:, :, None] == lo_iota).astype(jnp.float32)
        w = (match_lo * sel[:, :, None]).sum(axis=1)
        wt = jnp.broadcast_to(w[:, :, None], (B, P_REP, BS)).reshape(B, T_REP)

        qs = q_ref[...][:, 0]
        scale = D ** -0.5

        m_cols, l_cols, o_heads = [], [], []
        for h in range(H):
            if h + 1 < H:
                start_head_copy(h + 1)
            wait_head_copy(h)
            slot = h % 2
            kh = kh_buf[slot].reshape(T_REP, D)
            vh = vh_buf[slot].reshape(T_REP, D)
            q_h = qs[:, h, :]
            s = lax.dot_general(
                q_h, kh, (((1,), (1,)), ((), ())),
                preferred_element_type=jnp.float32) * scale
            m_h = s.max(axis=1, keepdims=True)
            e = jnp.exp(s - m_h) * wt
            l_h = e.sum(axis=1, keepdims=True)
            o_h = lax.dot_general(
                e, vh, (((1,), (0,)), ((), ())),
                preferred_element_type=jnp.float32)
            m_cols.append(m_h)
            l_cols.append(l_h)
            o_heads.append(o_h[:, None, :])

        m = jnp.concatenate(m_cols, axis=1)
        l = jnp.concatenate(l_cols, axis=1)
        o = jnp.concatenate(o_heads, axis=1)

        part_send[...] = jnp.concatenate(
            [o, m[:, :, None], l[:, :, None]], axis=2)

        pl.semaphore_wait(barrier, R)

        rdma_y = pltpu.make_async_remote_copy(
            src_ref=part_send, dst_ref=part_recv,
            send_sem=y_send_sem, recv_sem=y_recv_sem,
            device_id=partner, device_id_type=pl.DeviceIdType.MESH)
        rdma_y.start()
        rdma_y.wait_recv()

        m_p = part_recv[:, :, D]
        l_p = part_recv[:, :, D + 1]
        o_p = part_recv[:, :, :D]
        m2 = jnp.maximum(m, m_p)
        c_s = jnp.exp(m - m2)
        c_p = jnp.exp(m_p - m2)
        l2 = l * c_s + l_p * c_p
        o2 = o * c_s[:, :, None] + o_p * c_p[:, :, None]
        part_g[...] = jnp.concatenate(
            [o2, m2[:, :, None], l2[:, :, None]], axis=2)

        pltpu.make_async_copy(part_g, gather_buf.at[my_r], self_sem).start()
        for idx in range(R - 1):
            rr = lax.rem(my_r + 1 + idx, R)
            pltpu.make_async_remote_copy(
                src_ref=part_g, dst_ref=gather_buf.at[my_r],
                send_sem=g_send_sems.at[idx], recv_sem=g_recv_sems.at[my_r],
                device_id=(rr // 4, my_y, lax.rem(rr, 4)),
                device_id_type=pl.DeviceIdType.MESH).start()
        pltpu.make_async_copy(part_g, gather_buf.at[my_r], self_sem).wait()
        for idx in range(R - 1):
            rr = lax.rem(my_r + 1 + idx, R)
            pltpu.make_async_remote_copy(
                src_ref=part_g, dst_ref=gather_buf.at[rr],
                send_sem=g_send_sems.at[idx], recv_sem=g_recv_sems.at[rr],
                device_id=(rr // 4, my_y, lax.rem(rr, 4)),
                device_id_type=pl.DeviceIdType.MESH).wait_recv()

        parts = gather_buf[...]
        m_all = parts[:, :, :, D]
        l_all = parts[:, :, :, D + 1]
        o_all = parts[:, :, :, :D]
        m_g = m_all.max(axis=0)
        c = jnp.exp(m_all - m_g[None])
        l_g = (l_all * c).sum(axis=0)
        o_g = (o_all * c[:, :, :, None]).sum(axis=0)
        out_ref[...] = (o_g / l_g[:, :, None]).reshape(B, 1, H, D)

        rdma_y.wait_send()
        for idx in range(R - 1):
            rr = lax.rem(my_r + 1 + idx, R)
            pltpu.make_async_remote_copy(
                src_ref=part_g, dst_ref=gather_buf.at[my_r],
                send_sem=g_send_sems.at[idx], recv_sem=g_recv_sems.at[my_r],
                device_id=(rr // 4, my_y, lax.rem(rr, 4)),
                device_id_type=pl.DeviceIdType.MESH).wait_send()

    return pl.pallas_call(
        body,
        out_shape=jax.ShapeDtypeStruct((B, 1, H, D), jnp.float32),
        in_specs=[
            pl.BlockSpec(memory_space=pltpu.VMEM),
            pl.BlockSpec(memory_space=pltpu.MemorySpace.HBM),
            pl.BlockSpec(memory_space=pltpu.MemorySpace.HBM),
            pl.BlockSpec(memory_space=pltpu.VMEM),
            pl.BlockSpec(memory_space=pltpu.VMEM),
        ],
        out_specs=pl.BlockSpec(memory_space=pltpu.VMEM),
        scratch_shapes=[
            pltpu.VMEM((2, P_REP, BS, D), jnp.float32),
            pltpu.VMEM((2, P_REP, BS, D), jnp.float32),
            pltpu.VMEM((B, H, PK), jnp.float32),
            pltpu.VMEM((B, H, PK), jnp.float32),
            pltpu.VMEM((B, H, PK), jnp.float32),
            pltpu.VMEM((R, B, H, PK), jnp.float32),
            pltpu.SemaphoreType.DMA((2, 2)),
            pltpu.SemaphoreType.DMA,
            pltpu.SemaphoreType.DMA,
            pltpu.SemaphoreType.DMA((R - 1,)),
            pltpu.SemaphoreType.DMA((R,)),
            pltpu.SemaphoreType.DMA,
        ],
        compiler_params=pltpu.CompilerParams(collective_id=0),
    )(Q, K, V, bt, lens2)
